# baseline (speedup 1.0000x reference)
import functools

import jax
import jax.numpy as jnp
from jax import lax
from jax.experimental import pallas as pl
from jax.experimental.pallas import tpu as pltpu
from jax.experimental.pallas import tpu_sc as plsc

B, L, HIDDEN = 1024, 200, 128
N = B * L
NC, NS = 2, 16
NW = NC * NS
PER_W = N // NW
CHUNK = 128
NCHUNK = PER_W // CHUNK


def _sc_word_gather(word_table, ids_grp):
    @functools.partial(
        pl.kernel,
        mesh=plsc.VectorSubcoreMesh(core_axis_name="c", subcore_axis_name="s"),
        out_type=jax.ShapeDtypeStruct((N, HIDDEN), jnp.float32),
        scratch_types=[
            pltpu.VMEM((NCHUNK, CHUNK), jnp.int32),
            pltpu.VMEM((2, CHUNK, HIDDEN), jnp.float32),
            pltpu.SemaphoreType.DMA,
            pltpu.SemaphoreType.DMA,
        ],
    )
    def k(table_hbm, idx_hbm, out_hbm, idx_v, rows_v, sem0, sem1):
        wid = lax.axis_index("s") * NC + lax.axis_index("c")
        base = wid * PER_W
        pltpu.sync_copy(idx_hbm.at[wid], idx_v)
        pltpu.async_copy(table_hbm.at[idx_v.at[0]], rows_v.at[0], sem0)

        def body(i, carry):
            c0 = i * 2
            c1 = c0 + 1
            pltpu.async_copy(table_hbm.at[idx_v.at[c1]], rows_v.at[1], sem1)
            pltpu.make_async_copy(
                table_hbm.at[idx_v.at[c0]], rows_v.at[0], sem0).wait()
            pltpu.sync_copy(rows_v.at[0],
                            out_hbm.at[pl.ds(base + c0 * CHUNK, CHUNK)])

            @pl.when(c1 + 1 < NCHUNK)
            def _():
                pltpu.async_copy(
                    table_hbm.at[idx_v.at[c1 + 1]], rows_v.at[0], sem0)

            pltpu.make_async_copy(
                table_hbm.at[idx_v.at[c1]], rows_v.at[1], sem1).wait()
            pltpu.sync_copy(rows_v.at[1],
                            out_hbm.at[pl.ds(base + c1 * CHUNK, CHUNK)])
            return carry

        lax.fori_loop(0, NCHUNK // 2, body, 0)

    return k(word_table, ids_grp)


TBLK = 2048
NBLK = N // TBLK


def _tc_body(rows_ref, age_ref, seg_ref, pos_ref, aget_ref, segt_ref,
             post_ref, g_ref, b_ref, out_ref):
    x = rows_ref[...]
    age = age_ref[...]
    oh_a = (lax.broadcasted_iota(jnp.int32, (TBLK, 128), 1) == age)
    x = x + jnp.dot(oh_a.astype(jnp.bfloat16), aget_ref[...],
                    preferred_element_type=jnp.float32)
    pos = pos_ref[...]
    oh_p = (lax.broadcasted_iota(jnp.int32, (TBLK, 256), 1) == pos)
    x = x + jnp.dot(oh_p.astype(jnp.bfloat16), post_ref[...],
                    preferred_element_type=jnp.float32)
    seg = seg_ref[...]
    st = segt_ref[...]
    x = x + jnp.where(seg == 0, st[0:1, :], st[1:2, :])
    mean = jnp.mean(x, axis=1, keepdims=True)
    xc = x - mean
    var = jnp.mean(xc * xc, axis=1, keepdims=True)
    inv = lax.rsqrt(var + 1e-12)
    out_ref[...] = xc * inv * g_ref[...] + b_ref[...]


def _tc_finish(rows, age2, seg2, pos2, aget, segt, post, gamma2, beta2):
    return pl.pallas_call(
        _tc_body,
        grid=(NBLK,),
        in_specs=[
            pl.BlockSpec((TBLK, HIDDEN), lambda i: (i, 0)),
            pl.BlockSpec((TBLK, 1), lambda i: (i, 0)),
            pl.BlockSpec((TBLK, 1), lambda i: (i, 0)),
            pl.BlockSpec((TBLK, 1), lambda i: (i, 0)),
            pl.BlockSpec((128, HIDDEN), lambda i: (0, 0)),
            pl.BlockSpec((8, HIDDEN), lambda i: (0, 0)),
            pl.BlockSpec((256, HIDDEN), lambda i: (0, 0)),
            pl.BlockSpec((1, HIDDEN), lambda i: (0, 0)),
            pl.BlockSpec((1, HIDDEN), lambda i: (0, 0)),
        ],
        out_specs=pl.BlockSpec((TBLK, HIDDEN), lambda i: (i, 0)),
        out_shape=jax.ShapeDtypeStruct((N, HIDDEN), jnp.float32),
        compiler_params=pltpu.CompilerParams(
            dimension_semantics=("arbitrary",)),
    )(rows, age2, seg2, pos2, aget, segt, post, gamma2, beta2)


def kernel(word_ids, age_ids, seg_ids, posi_ids, word_table, seg_table,
           age_table, posi_table, gamma, beta):
    ids_grp = word_ids.astype(jnp.int32).reshape(NW, NCHUNK, CHUNK)
    rows = _sc_word_gather(word_table, ids_grp)

    age2 = age_ids.astype(jnp.int32).reshape(N, 1)
    seg2 = seg_ids.astype(jnp.int32).reshape(N, 1)
    pos2 = posi_ids.astype(jnp.int32).reshape(N, 1)
    aget = jnp.pad(age_table, ((0, 128 - age_table.shape[0]),
                               (0, 0))).astype(jnp.bfloat16)
    segt = jnp.pad(seg_table, ((0, 8 - seg_table.shape[0]), (0, 0)))
    post = posi_table[:256].astype(jnp.bfloat16)
    out = _tc_finish(rows, age2, seg2, pos2, aget, segt, post,
                     gamma.reshape(1, HIDDEN), beta.reshape(1, HIDDEN))
    return out.reshape(B, L, HIDDEN)


# speedup vs baseline: 1.7037x; 1.7037x over previous
import functools

import jax
import jax.numpy as jnp
from jax import lax
from jax.experimental import pallas as pl
from jax.experimental.pallas import tpu as pltpu
from jax.experimental.pallas import tpu_sc as plsc

B, L, HIDDEN = 1024, 200, 128
N = B * L
NC, NS = 2, 16
NW = NC * NS
PER_W = N // NW
CHUNK = 128
NCHUNK = PER_W // CHUNK


def _sc_word_gather(word_table, ids_grp):
    @functools.partial(
        pl.kernel,
        mesh=plsc.VectorSubcoreMesh(core_axis_name="c", subcore_axis_name="s"),
        out_type=jax.ShapeDtypeStruct((N, HIDDEN), jnp.float32),
        scratch_types=[
            pltpu.VMEM((NCHUNK, CHUNK), jnp.int32),
            pltpu.VMEM((2, CHUNK, HIDDEN), jnp.float32),
            pltpu.SemaphoreType.DMA,
            pltpu.SemaphoreType.DMA,
        ],
    )
    def k(table_hbm, idx_hbm, out_hbm, idx_v, rows_v, sem0, sem1):
        wid = lax.axis_index("s") * NC + lax.axis_index("c")
        base = wid * PER_W
        pltpu.sync_copy(idx_hbm.at[wid], idx_v)
        pltpu.async_copy(table_hbm.at[idx_v.at[0]], rows_v.at[0], sem0)

        def body(i, carry):
            c0 = i * 2
            c1 = c0 + 1
            pltpu.async_copy(table_hbm.at[idx_v.at[c1]], rows_v.at[1], sem1)
            pltpu.make_async_copy(
                table_hbm.at[idx_v.at[c0]], rows_v.at[0], sem0).wait()
            pltpu.sync_copy(rows_v.at[0],
                            out_hbm.at[pl.ds(base + c0 * CHUNK, CHUNK)])

            @pl.when(c1 + 1 < NCHUNK)
            def _():
                pltpu.async_copy(
                    table_hbm.at[idx_v.at[c1 + 1]], rows_v.at[0], sem0)

            pltpu.make_async_copy(
                table_hbm.at[idx_v.at[c1]], rows_v.at[1], sem1).wait()
            pltpu.sync_copy(rows_v.at[1],
                            out_hbm.at[pl.ds(base + c1 * CHUNK, CHUNK)])
            return carry

        lax.fori_loop(0, NCHUNK // 2, body, 0)

    return k(word_table, ids_grp)


TBLK = 2048
NBLK = N // TBLK


_DN_T = (((0,), (0,)), ((), ()))


def _tc_body(rows_ref, age_ref, seg_ref, pos_ref, aget_ref, segt_ref,
             post_ref, g_ref, b_ref, out_ref):
    x = rows_ref[...]
    age = age_ref[0]
    oh_a = (lax.broadcasted_iota(jnp.int32, (128, TBLK), 0) == age)
    x = x + lax.dot_general(oh_a.astype(jnp.bfloat16), aget_ref[...], _DN_T,
                            preferred_element_type=jnp.float32)
    pos = pos_ref[0]
    oh_p = (lax.broadcasted_iota(jnp.int32, (256, TBLK), 0) == pos)
    x = x + lax.dot_general(oh_p.astype(jnp.bfloat16), post_ref[...], _DN_T,
                            preferred_element_type=jnp.float32)
    seg = seg_ref[0]
    oh_s = (lax.broadcasted_iota(jnp.int32, (8, TBLK), 0) == seg)
    x = x + lax.dot_general(oh_s.astype(jnp.bfloat16), segt_ref[...], _DN_T,
                            preferred_element_type=jnp.float32)
    mean = jnp.mean(x, axis=1, keepdims=True)
    xc = x - mean
    var = jnp.mean(xc * xc, axis=1, keepdims=True)
    inv = lax.rsqrt(var + 1e-12)
    out_ref[...] = xc * inv * g_ref[...] + b_ref[...]


def _tc_finish(rows, age2, seg2, pos2, aget, segt, post, gamma2, beta2):
    return pl.pallas_call(
        _tc_body,
        grid=(NBLK,),
        in_specs=[
            pl.BlockSpec((TBLK, HIDDEN), lambda i: (i, 0)),
            pl.BlockSpec((1, 1, TBLK), lambda i: (i, 0, 0)),
            pl.BlockSpec((1, 1, TBLK), lambda i: (i, 0, 0)),
            pl.BlockSpec((1, 1, TBLK), lambda i: (i, 0, 0)),
            pl.BlockSpec((128, HIDDEN), lambda i: (0, 0)),
            pl.BlockSpec((8, HIDDEN), lambda i: (0, 0)),
            pl.BlockSpec((256, HIDDEN), lambda i: (0, 0)),
            pl.BlockSpec((1, HIDDEN), lambda i: (0, 0)),
            pl.BlockSpec((1, HIDDEN), lambda i: (0, 0)),
        ],
        out_specs=pl.BlockSpec((TBLK, HIDDEN), lambda i: (i, 0)),
        out_shape=jax.ShapeDtypeStruct((N, HIDDEN), jnp.float32),
        compiler_params=pltpu.CompilerParams(
            dimension_semantics=("arbitrary",)),
    )(rows, age2, seg2, pos2, aget, segt, post, gamma2, beta2)


def kernel(word_ids, age_ids, seg_ids, posi_ids, word_table, seg_table,
           age_table, posi_table, gamma, beta):
    ids_grp = word_ids.astype(jnp.int32).reshape(NW, NCHUNK, CHUNK)
    rows = _sc_word_gather(word_table, ids_grp)

    age2 = age_ids.astype(jnp.int32).reshape(NBLK, 1, TBLK)
    seg2 = seg_ids.astype(jnp.int32).reshape(NBLK, 1, TBLK)
    pos2 = posi_ids.astype(jnp.int32).reshape(NBLK, 1, TBLK)
    aget = jnp.pad(age_table, ((0, 128 - age_table.shape[0]),
                               (0, 0))).astype(jnp.bfloat16)
    segt = jnp.pad(seg_table, ((0, 8 - seg_table.shape[0]),
                               (0, 0))).astype(jnp.bfloat16)
    post = posi_table[:256].astype(jnp.bfloat16)
    out = _tc_finish(rows, age2, seg2, pos2, aget, segt, post,
                     gamma.reshape(1, HIDDEN), beta.reshape(1, HIDDEN))
    return out.reshape(B, L, HIDDEN)
